# baseline (speedup 1.0000x reference)
import functools

import jax
import jax.numpy as jnp
from jax import lax
from jax.experimental import pallas as pl
from jax.experimental.pallas import tpu as pltpu
from jax.experimental.pallas import tpu_sc as plsc

NC, NS, L = 2, 16, 16
NW = NC * NS
B, T = 16384, 200
OUT_D = 15
BW = B // NW
HB = 256
TR = 8
NT = T // TR

MB, DB, SB, TB = 0, 48, 69, 369
TAB_PAD = 416

_mesh = plsc.VectorSubcoreMesh(core_axis_name="c", subcore_axis_name="s")

_IDX_BUF = pltpu.VMEM((TR, HB), jnp.int32)
_OUT_BUF = pltpu.VMEM((OUT_D, TR, HB), jnp.float32)


@functools.partial(
    pl.kernel,
    out_type=jax.ShapeDtypeStruct((OUT_D, T, B), jnp.float32),
    mesh=_mesh,
    compiler_params=pltpu.CompilerParams(needs_layout_passes=False),
    scratch_types=[
        pltpu.VMEM((TAB_PAD,), jnp.float32),
        _IDX_BUF, _IDX_BUF, _IDX_BUF, _IDX_BUF,
        _IDX_BUF, _IDX_BUF, _IDX_BUF, _IDX_BUF,
        _OUT_BUF, _OUT_BUF,
        pltpu.SemaphoreType.DMA, pltpu.SemaphoreType.DMA,
        pltpu.SemaphoreType.DMA, pltpu.SemaphoreType.DMA,
    ],
)
def _emb_lookup(mi, di, si, ti, tab, out_hbm, tab_v,
                mi_a, di_a, si_a, ti_a, mi_b, di_b, si_b, ti_b,
                out_a, out_b, isem_a, isem_b, osem_a, osem_b):
    wid = lax.axis_index("s") * NC + lax.axis_index("c")
    b0 = wid * BW
    pltpu.sync_copy(tab, tab_v)
    lanes = lax.iota(jnp.int32, L)
    tm_v = [plsc.load_gather(tab_v, [lanes * 4 + (MB + j)]) for j in range(4)]
    td_v = [plsc.load_gather(tab_v, [lanes * 3 + (DB + j)]) for j in range(3)]
    tt_v = [plsc.load_gather(tab_v, [lanes * 2 + (TB + j)]) for j in range(2)]

    def vperm(tbl, idx):
        return jax.lax.gather(
            tbl, idx[:, None],
            jax.lax.GatherDimensionNumbers(
                offset_dims=(), collapsed_slice_dims=(0,),
                start_index_map=(0,)),
            slice_sizes=(1,),
            mode=jax.lax.GatherScatterMode.PROMISE_IN_BOUNDS)

    bufs = ((mi_a, di_a, si_a, ti_a, out_a, isem_a, osem_a),
            (mi_b, di_b, si_b, ti_b, out_b, isem_b, osem_b))

    def in_slices(tt, h):
        r0 = tt * TR
        bh = b0 + h * HB
        return [src.at[pl.ds(r0, TR), pl.ds(bh, HB)]
                for src in (mi, di, si, ti)]

    def start_in(tt, h):
        bm, bd, bs, bt, _, isem, _ = bufs[h]
        for src, dst in zip(in_slices(tt, h), (bm, bd, bs, bt)):
            pltpu.async_copy(src, dst, isem)

    def wait_in(tt, h):
        bm, bd, bs, bt, _, isem, _ = bufs[h]
        for src, dst in zip(in_slices(tt, h), (bm, bd, bs, bt)):
            pltpu.make_async_copy(src, dst, isem).wait()

    def out_slice(tt, h):
        return out_hbm.at[:, pl.ds(tt * TR, TR), pl.ds(b0 + h * HB, HB)]

    def start_out(tt, h):
        ov, osem = bufs[h][4], bufs[h][6]
        pltpu.async_copy(ov, out_slice(tt, h), osem)

    def wait_out(tt, h):
        ov, osem = bufs[h][4], bufs[h][6]
        pltpu.make_async_copy(ov, out_slice(tt, h), osem).wait()

    def compute(h):
        bm, bd, bs, bt, ov = bufs[h][:5]

        @plsc.parallel_loop(0, HB, step=L, unroll=1)
        def group(g):
            for r in range(TR):
                m = bm[r, pl.ds(g, L)]
                d = bd[r, pl.ds(g, L)]
                s = bs[r, pl.ds(g, L)]
                t = bt[r, pl.ds(g, L)]
                vals = [vperm(tm_v[j], m) for j in range(4)]
                vals += [vperm(td_v[j], d) for j in range(3)]
                vals += [plsc.load_gather(tab_v, [s * 6 + (SB + j)])
                         for j in range(6)]
                vals += [vperm(tt_v[j], t) for j in range(2)]
                for f, v in enumerate(vals):
                    ov[f, r, pl.ds(g, L)] = v

    start_in(0, 0)

    def t_step(tt, carry):
        start_in(tt, 1)
        wait_in(tt, 0)

        @pl.when(tt > 0)
        def _():
            wait_out(tt, 0)

        compute(0)
        start_out(tt, 0)

        @pl.when(tt + 1 < NT)
        def _():
            start_in(tt + 1, 0)

        wait_in(tt, 1)

        @pl.when(tt > 0)
        def _():
            wait_out(tt, 1)

        compute(1)
        start_out(tt, 1)
        return carry

    lax.fori_loop(0, NT, t_step, 0, unroll=False)
    wait_out(NT - 1, 0)
    wait_out(NT - 1, 1)


def kernel(month_idx, day_idx, sp_idx, dtype_idx, emb_month, emb_day, emb_sp,
           emb_dtype):
    mi = month_idx.astype(jnp.int32).T
    di = day_idx.astype(jnp.int32).T
    si = sp_idx.astype(jnp.int32).T
    ti = dtype_idx.astype(jnp.int32).T
    tab = jnp.concatenate([
        emb_month.reshape(-1),
        emb_day.reshape(-1),
        emb_sp.reshape(-1),
        emb_dtype.reshape(-1),
        jnp.zeros((TAB_PAD - 373,), jnp.float32),
    ])
    out = _emb_lookup(mi, di, si, ti, tab)
    return out.transpose(2, 1, 0)
